# baseline (device time: 27495 ns/iter reference)
import jax
import jax.numpy as jnp
from jax import lax
from jax.experimental import pallas as pl
from jax.experimental.pallas import tpu as pltpu

N_CHUNKS = 8


def kernel(x, W, labels):
    T, D = x.shape
    _, V = W.shape
    VC = V // N_CHUNKS

    def body(
        x_ref, w_ref, lbl_ref, out_ref,
        lbuf, s128, g128, comm_ref, send_sem, recv_sem,
    ):
        c = pl.program_id(0)
        my_x = lax.axis_index("x")
        my_y = lax.axis_index("y")
        my_z = lax.axis_index("z")
        partner = (1 - my_x, my_y, my_z)
        par = c % 2

        @pl.when(c == 0)
        def _():
            s128[:, :] = jnp.zeros((T, 128), jnp.float32)
            g128[:, :] = jnp.zeros((T, 128), jnp.float32)

        def stats(buf):
            l = buf[:, :]
            ones = jnp.ones((VC, 128), jnp.float32)
            E = jnp.exp(l)
            s128[:, :] = s128[:, :] + jnp.dot(
                E, ones, preferred_element_type=jnp.float32
            )
            lbl_local = lbl_ref[:] - (my_x * V + (c - 1) * VC)
            col = lax.broadcasted_iota(jnp.int32, (T, VC), 1)
            Mk = jnp.where(col == lbl_local[:, None], l, 0.0)
            g128[:, :] = g128[:, :] + jnp.dot(
                Mk, ones, preferred_element_type=jnp.float32
            )

        @pl.when(jnp.logical_and(c > 0, par == 1))
        def _():
            stats(lbuf.at[0])

        @pl.when(jnp.logical_and(c > 0, par == 0))
        def _():
            stats(lbuf.at[1])

        def mm(buf):
            buf[:, :] = jnp.dot(
                x_ref[:, :], w_ref[:, :], preferred_element_type=jnp.float32
            )

        @pl.when(jnp.logical_and(c < N_CHUNKS, par == 0))
        def _():
            mm(lbuf.at[0])

        @pl.when(jnp.logical_and(c < N_CHUNKS, par == 1))
        def _():
            mm(lbuf.at[1])

        @pl.when(c == N_CHUNKS)
        def _():
            s_loc = jnp.sum(s128[:, :], axis=1) * (1.0 / 128.0)
            g_loc = jnp.sum(g128[:, :], axis=1) * (1.0 / 128.0)

            barrier_sem = pltpu.get_barrier_semaphore()
            pl.semaphore_signal(
                barrier_sem, inc=1, device_id=partner,
                device_id_type=pl.DeviceIdType.MESH,
            )
            pl.semaphore_wait(barrier_sem, 1)

            comm_ref[0, 0, :] = s_loc
            comm_ref[0, 1, :] = g_loc
            rdma = pltpu.make_async_remote_copy(
                src_ref=comm_ref.at[0],
                dst_ref=comm_ref.at[1],
                send_sem=send_sem,
                recv_sem=recv_sem,
                device_id=partner,
                device_id_type=pl.DeviceIdType.MESH,
            )
            rdma.start()
            rdma.wait()

            s_tot = s_loc + comm_ref[1, 0, :]
            g_tot = g_loc + comm_ref[1, 1, :]
            out_ref[:] = jnp.log(s_tot) - g_tot

    return pl.pallas_call(
        body,
        grid=(N_CHUNKS + 1,),
        out_shape=jax.ShapeDtypeStruct((T,), jnp.float32),
        in_specs=[
            pl.BlockSpec((T, D), lambda c: (0, 0)),
            pl.BlockSpec((D, VC), lambda c: (0, jnp.minimum(c, N_CHUNKS - 1))),
            pl.BlockSpec((T,), lambda c: (0,)),
        ],
        out_specs=pl.BlockSpec((T,), lambda c: (0,)),
        scratch_shapes=[
            pltpu.VMEM((2, T, VC), jnp.float32),
            pltpu.VMEM((T, 128), jnp.float32),
            pltpu.VMEM((T, 128), jnp.float32),
            pltpu.VMEM((2, 2, T), jnp.float32),
            pltpu.SemaphoreType.DMA,
            pltpu.SemaphoreType.DMA,
        ],
        compiler_params=pltpu.CompilerParams(
            collective_id=0,
            vmem_limit_bytes=100 * 1024 * 1024,
            dimension_semantics=("arbitrary",),
        ),
    )(x, W, labels)


# device time: 23261 ns/iter; 1.1820x vs baseline; 1.1820x over previous
import jax
import jax.numpy as jnp
from jax import lax
from jax.experimental import pallas as pl
from jax.experimental.pallas import tpu as pltpu

N_CHUNKS = 8


def kernel(x, W, labels):
    T, D = x.shape
    _, V = W.shape
    VC = V // N_CHUNKS

    def body(
        x_ref, w_hbm, lbl_ref, out_ref,
        wv, comm_ref, copy_sems, send_sem, recv_sem,
    ):
        my_x = lax.axis_index("x")
        my_y = lax.axis_index("y")
        my_z = lax.axis_index("z")
        partner = (1 - my_x, my_y, my_z)

        barrier_sem = pltpu.get_barrier_semaphore()
        pl.semaphore_signal(
            barrier_sem, inc=1, device_id=partner,
            device_id_type=pl.DeviceIdType.MESH,
        )

        copies = []
        for c in range(N_CHUNKS):
            cp = pltpu.make_async_copy(
                w_hbm.at[:, pl.ds(c * VC, VC)],
                wv.at[:, pl.ds(c * VC, VC)],
                copy_sems.at[c],
            )
            cp.start()
            copies.append(cp)

        xv = x_ref[:, :]
        lbl = lbl_ref[:]
        s_loc = None
        g_loc = None
        for c in range(N_CHUNKS):
            copies[c].wait()
            logits_c = jnp.dot(
                xv, wv[:, c * VC:(c + 1) * VC],
                preferred_element_type=jnp.float32,
            )
            e = jnp.sum(jnp.exp(logits_c), axis=1)
            lbl_local = lbl - (my_x * V + c * VC)
            col = lax.broadcasted_iota(jnp.int32, (T, VC), 1)
            gp = jnp.sum(
                jnp.where(col == lbl_local[:, None], logits_c, 0.0), axis=1
            )
            s_loc = e if s_loc is None else s_loc + e
            g_loc = gp if g_loc is None else g_loc + gp

        pl.semaphore_wait(barrier_sem, 1)
        comm_ref[0, 0, :] = s_loc
        comm_ref[0, 1, :] = g_loc
        rdma = pltpu.make_async_remote_copy(
            src_ref=comm_ref.at[0],
            dst_ref=comm_ref.at[1],
            send_sem=send_sem,
            recv_sem=recv_sem,
            device_id=partner,
            device_id_type=pl.DeviceIdType.MESH,
        )
        rdma.start()
        rdma.wait()

        s_tot = s_loc + comm_ref[1, 0, :]
        g_tot = g_loc + comm_ref[1, 1, :]
        out_ref[:] = jnp.log(s_tot) - g_tot

    return pl.pallas_call(
        body,
        out_shape=jax.ShapeDtypeStruct((T,), jnp.float32),
        in_specs=[
            pl.BlockSpec(memory_space=pltpu.VMEM),
            pl.BlockSpec(memory_space=pl.ANY),
            pl.BlockSpec(memory_space=pltpu.VMEM),
        ],
        out_specs=pl.BlockSpec(memory_space=pltpu.VMEM),
        scratch_shapes=[
            pltpu.VMEM((D, V), jnp.float32),
            pltpu.VMEM((2, 2, T), jnp.float32),
            pltpu.SemaphoreType.DMA((N_CHUNKS,)),
            pltpu.SemaphoreType.DMA,
            pltpu.SemaphoreType.DMA,
        ],
        compiler_params=pltpu.CompilerParams(
            collective_id=0,
            vmem_limit_bytes=100 * 1024 * 1024,
        ),
    )(x, W, labels)
